# baseline (speedup 1.0000x reference)
import functools

import numpy as np
import jax
import jax.numpy as jnp
from jax import lax
from jax.experimental import pallas as pl
from jax.experimental.pallas import tpu as pltpu
from jax.experimental.pallas import tpu_sc as plsc

_VOCAB = 100000
_N_TABLES = 26
_EMBED = 32
_B = 4096
_N_FEAT = 1 + _N_TABLES
_DI_DIM = _N_FEAT * (_N_FEAT + 1) // 2


_NC, _NS = 2, 16
_NW = _NC * _NS
_TOTAL = _B * _N_TABLES
_BPW = _TOTAL // _NW
_CHUNK = 128
_NCHUNK = _BPW // _CHUNK


_RM_ROWS = (_N_TABLES * _VOCAB) // 4
_NOB = _RM_ROWS // 64
_OB_TAIL = _RM_ROWS - _NOB * 64


def _sc_pack(table):
    mesh = plsc.VectorSubcoreMesh(core_axis_name="c", subcore_axis_name="s")

    @functools.partial(
        pl.kernel,
        mesh=mesh,
        out_type=jax.ShapeDtypeStruct((_RM_ROWS, 128), jnp.float32),
        scratch_types=[
            pltpu.VMEM((256, 32), jnp.float32),
            pltpu.VMEM((256, 32), jnp.float32),
            pltpu.VMEM((64, 128), jnp.float32),
            pltpu.VMEM((64, 128), jnp.float32),
            pltpu.SemaphoreType.DMA,
            pltpu.SemaphoreType.DMA,
        ],
        compiler_params=pltpu.CompilerParams(needs_layout_passes=False),
    )
    def k(tbl_hbm, out_hbm, s0, s1, o0, o1, isem, osem):
        wid = lax.axis_index("s") * _NC + lax.axis_index("c")

        def ob(kk):
            return wid + _NW * kk

        def fire_in(kk, sbuf):
            pltpu.async_copy(
                tbl_hbm.at[pl.ds(ob(kk) * 256, 256)], sbuf, isem)

        def drain(sem, buf):
            pltpu.make_async_copy(
                tbl_hbm.at[pl.ds(0, 256)] if buf.shape == (256, 32)
                else out_hbm.at[pl.ds(0, 64)], buf, sem).wait()

        def pack(sbuf, obuf):
            def pbody(p, carry):
                for g in range(8):
                    obuf[p, g * 16:(g + 1) * 16] = (
                        sbuf[4 * p + g // 2, (g % 2) * 16:(g % 2) * 16 + 16])
                return carry

            lax.fori_loop(0, 64, pbody, 0)

        def step(kk, sbuf, obuf, first):
            nxt = s1 if sbuf is s0 else s0

            @pl.when(ob(kk + 1) < _NOB)
            def _():
                fire_in(kk + 1, nxt)

            @pl.when(ob(kk) < _NOB)
            def _():
                drain(isem, sbuf)
                pack(sbuf, obuf)
                if not first:
                    drain(osem, obuf)
                pltpu.async_copy(obuf, out_hbm.at[pl.ds(ob(kk) * 64, 64)], osem)

        fire_in(0, s0)
        step(0, s0, o0, first=True)
        step(1, s1, o1, first=True)

        def body(k2, carry):
            step(2 * k2, s0, o0, first=False)
            step(2 * k2 + 1, s1, o1, first=False)
            return carry

        lax.fori_loop(1, 160, body, 0)
        drain(osem, o0)
        drain(osem, o1)

        @pl.when(wid == _NOB % _NW)
        def _():
            pltpu.sync_copy(
                tbl_hbm.at[pl.ds(_NOB * 256, _OB_TAIL * 4)],
                s0.at[pl.ds(0, _OB_TAIL * 4)])

            def pbody(p, carry):
                for g in range(8):
                    o0[p, g * 16:(g + 1) * 16] = (
                        s0[4 * p + g // 2, (g % 2) * 16:(g % 2) * 16 + 16])
                return carry

            lax.fori_loop(0, _OB_TAIL, pbody, 0)
            pltpu.sync_copy(
                o0.at[pl.ds(0, _OB_TAIL)],
                out_hbm.at[pl.ds(_NOB * 64, _OB_TAIL)])

    return k(table)


def _sc_gather(table_rm, idx3d):
    mesh = plsc.VectorSubcoreMesh(core_axis_name="c", subcore_axis_name="s")

    @functools.partial(
        pl.kernel,
        mesh=mesh,
        out_type=jax.ShapeDtypeStruct((_TOTAL, 128), jnp.float32),
        scratch_types=[
            pltpu.VMEM((_NCHUNK, _CHUNK), jnp.int32),
            pltpu.VMEM((_CHUNK, 128), jnp.float32),
            pltpu.VMEM((_CHUNK, 128), jnp.float32),
            pltpu.SemaphoreType.DMA,
        ],
        compiler_params=pltpu.CompilerParams(needs_layout_passes=False),
    )
    def k(table_hbm, idx_hbm, out_hbm, idx_v, g0, g1, sem):
        wid = lax.axis_index("s") * _NC + lax.axis_index("c")
        base = wid * _BPW
        pltpu.sync_copy(idx_hbm.at[wid], idx_v)

        def fire(c, gbuf):
            pltpu.async_copy(table_hbm.at[idx_v.at[c]], gbuf, sem)

        def drain(gbuf):
            pltpu.make_async_copy(
                table_hbm.at[pl.ds(0, _CHUNK)], gbuf, sem).wait()

        def flush(c, gbuf):
            pltpu.sync_copy(
                gbuf, out_hbm.at[pl.ds(base + c * _CHUNK, _CHUNK)])

        fire(0, g0)

        def body(c2, carry):
            c = 2 * c2

            @pl.when(c + 1 < _NCHUNK)
            def _():
                fire(c + 1, g1)

            drain(g0)
            flush(c, g0)

            @pl.when(c + 2 < _NCHUNK)
            def _():
                fire(c + 2, g0)

            @pl.when(c + 1 < _NCHUNK)
            def _():
                drain(g1)
                flush(c + 1, g1)
            return carry

        lax.fori_loop(0, (_NCHUNK + 1) // 2, body, 0)

    return k(table_rm, idx3d)


_BLK = 256
_GRID = _B // _BLK


def _dense_body(dense_ref, embed_ref, li_ref,
                bw0, bb0, bw1, bb1, bw2, bb2,
                tw0, tb0, tw1, tb1, tw2, tb2, tw3, tb3, tw4, tb4,
                out_ref, acc_ref):
    h = dense_ref[:]
    h = jnp.maximum(h @ bw0[:] + bb0[:], 0.0)
    h = jnp.maximum(h @ bw1[:] + bb1[:], 0.0)
    bot = jnp.maximum(h @ bw2[:] + bb2[:], 0.0)

    li = li_ref[:][:, :, None]
    emb = jnp.where(li == 0, embed_ref[:, :, 0:32], 0.0)
    emb = emb + jnp.where(li == 1, embed_ref[:, :, 32:64], 0.0)
    emb = emb + jnp.where(li == 2, embed_ref[:, :, 64:96], 0.0)
    emb = emb + jnp.where(li == 3, embed_ref[:, :, 96:128], 0.0)
    feat = jnp.concatenate([bot.reshape(_BLK, 1, _EMBED), emb], axis=1)
    xact = jax.lax.dot_general(
        feat, feat, (((2,), (2,)), ((0,), (0,))),
        preferred_element_type=jnp.float32)
    acc_ref[:, 0:_EMBED] = bot
    off = _EMBED
    for i in range(_N_FEAT):
        w = _N_FEAT - i
        acc_ref[:, off:off + w] = xact[:, i, i:]
        off += w

    x = acc_ref[:]
    x = jnp.maximum(x @ tw0[:] + tb0[:], 0.0)
    x = jnp.maximum(x @ tw1[:] + tb1[:], 0.0)
    x = jnp.maximum(x @ tw2[:] + tb2[:], 0.0)
    x = jnp.maximum(x @ tw3[:] + tb3[:], 0.0)
    x = x @ tw4[:] + tb4[:]
    out_ref[:] = jax.nn.sigmoid(x)


def _dense_call(dense, embed, li, bw0, bb0, bw1, bb1, bw2, bb2,
                tw0, tb0, tw1, tb1, tw2, tb2, tw3, tb3, tw4, tb4):
    def full(a):
        return pl.BlockSpec(a.shape, lambda i: (0,) * a.ndim)

    ws = (bw0, bb0, bw1, bb1, bw2, bb2,
          tw0, tb0, tw1, tb1, tw2, tb2, tw3, tb3, tw4, tb4)
    return pl.pallas_call(
        _dense_body,
        grid=(_GRID,),
        in_specs=[
            pl.BlockSpec((_BLK, dense.shape[1]), lambda i: (i, 0)),
            pl.BlockSpec((_BLK, _N_TABLES, 128), lambda i: (i, 0, 0)),
            pl.BlockSpec((_BLK, _N_TABLES), lambda i: (i, 0)),
        ] + [full(w) for w in ws],
        out_specs=pl.BlockSpec((_BLK, 1), lambda i: (i, 0)),
        out_shape=jax.ShapeDtypeStruct((_B, 1), jnp.float32),
        scratch_shapes=[pltpu.VMEM((_BLK, _EMBED + _DI_DIM), jnp.float32)],
        compiler_params=pltpu.CompilerParams(
            dimension_semantics=("arbitrary",),
        ),
    )(dense, embed, li, *ws)


def kernel(dense_features, cat_features, embedding_table,
           bw0, bb0, bw1, bb1, bw2, bb2,
           tw0, tb0, tw1, tb1, tw2, tb2, tw3, tb3, tw4, tb4):
    offsets = jnp.asarray(np.arange(_N_TABLES, dtype=np.int32) * _VOCAB)
    idx = cat_features + offsets[None, :]
    idx4 = (idx >> 2).reshape(_NW, _NCHUNK, _CHUNK)
    li = idx & 3
    table_rm = _sc_pack(embedding_table)
    rows = _sc_gather(table_rm, idx4)
    embed = rows.reshape(_B, _N_TABLES, 128)
    b2 = lambda v: v.reshape(1, -1)
    return _dense_call(dense_features, embed, li,
                       bw0, b2(bb0), bw1, b2(bb1), bw2, b2(bb2),
                       tw0, b2(tb0), tw1, b2(tb1), tw2, b2(tb2),
                       tw3, b2(tb3), tw4, b2(tb4))
